# Initial kernel scaffold; baseline (speedup 1.0000x reference)
import jax
import jax.numpy as jnp
from jax.experimental import pallas as pl


def kernel(x_web, x_usr, edge_clicks, edge_rev, lin_in_W, lin_in_b, Wk, bk, Wq, bq, Wv, bv, Wa, ba, skip, a_rel, m_rel, p_rel, lin_out_W, lin_out_b):
    raise NotImplementedError("write your pallas kernel here")


# speedup vs baseline: 10.0914x; 10.0914x over previous
import functools
import math

import jax
import jax.numpy as jnp
from jax import lax
from jax.experimental import pallas as pl
from jax.experimental.pallas import tpu as pltpu
from jax.experimental.pallas import tpu_sc as plsc

N = 50000
F_IN = 128
HID = 64
H = 4
D = 16
L = 2
C = 16
E = 600000

NC = 2
NS = 16
LANES = 16

HP = 2
W_ROW = 34
CHUNK = 128
E_PAD = 600064
N_ACC = 50048
TRASH = N
BN = 1000
GRID = N // BN

_f32 = jnp.float32


def _inproj_body(xw_ref, xu_ref, w_ref, b_ref, h0_ref, h1_ref):
    w = w_ref[...]
    b = b_ref[...]
    h0_ref[...] = jnp.maximum(
        jnp.dot(xw_ref[...], w[0], precision=lax.Precision.HIGHEST) + b[0:1, :], 0.0)
    h1_ref[...] = jnp.maximum(
        jnp.dot(xu_ref[...], w[1], precision=lax.Precision.HIGHEST) + b[1:2, :], 0.0)


def _inproj(x_web, x_usr, lin_in_W, lin_in_b):
    return pl.pallas_call(
        _inproj_body,
        grid=(GRID,),
        in_specs=[
            pl.BlockSpec((BN, F_IN), lambda i: (i, 0)),
            pl.BlockSpec((BN, F_IN), lambda i: (i, 0)),
            pl.BlockSpec((2, F_IN, HID), lambda i: (0, 0, 0)),
            pl.BlockSpec((2, HID), lambda i: (0, 0)),
        ],
        out_specs=[
            pl.BlockSpec((BN, HID), lambda i: (i, 0)),
            pl.BlockSpec((BN, HID), lambda i: (i, 0)),
        ],
        out_shape=[jax.ShapeDtypeStruct((N, HID), _f32)] * 2,
    )(x_web, x_usr, lin_in_W, lin_in_b)


def _qkv_body(h0_ref, h1_ref, w0_ref, b0_ref, w1_ref, b1_ref,
              qd0_ref, kt1_ref, vt1_ref, qd1_ref, kt0_ref, vt0_ref):
    out0 = jnp.dot(h0_ref[...], w0_ref[...],
                   precision=lax.Precision.HIGHEST) + b0_ref[...]
    out1 = jnp.dot(h1_ref[...], w1_ref[...],
                   precision=lax.Precision.HIGHEST) + b1_ref[...]
    for hp in range(HP):
        s = hp * 32
        qd0_ref[hp, :, :] = out0[:, s:s + 32]
        kt1_ref[hp, :, :] = out0[:, 64 + s:64 + s + 32]
        vt1_ref[hp, :, :] = out0[:, 128 + s:128 + s + 32]
        qd1_ref[hp, :, :] = out1[:, s:s + 32]
        kt0_ref[hp, :, :] = out1[:, 64 + s:64 + s + 32]
        vt0_ref[hp, :, :] = out1[:, 128 + s:128 + s + 32]


def _qkv(h0, h1, w0cat, b0cat, w1cat, b1cat):
    outs = pl.pallas_call(
        _qkv_body,
        grid=(GRID,),
        in_specs=[
            pl.BlockSpec((BN, HID), lambda i: (i, 0)),
            pl.BlockSpec((BN, HID), lambda i: (i, 0)),
            pl.BlockSpec((HID, 192), lambda i: (0, 0)),
            pl.BlockSpec((1, 192), lambda i: (0, 0)),
            pl.BlockSpec((HID, 192), lambda i: (0, 0)),
            pl.BlockSpec((1, 192), lambda i: (0, 0)),
        ],
        out_specs=[pl.BlockSpec((HP, BN, 32), lambda i: (0, i, 0))] * 6,
        out_shape=[jax.ShapeDtypeStruct((HP, N, 32), _f32)] * 6,
    )(h0, h1, w0cat, b0cat, w1cat, b1cat)
    return outs


def _epilogue_body(h0_ref, h1_ref, nd0_ref, nd1_ref, wa_ref, ba_ref, sgm_ref,
                   o0_ref, o1_ref):
    wa = wa_ref[...]
    ba = ba_ref[...]
    sgm = sgm_ref[...]
    for t, (nd_ref, h_ref, o_ref) in enumerate(
            [(nd0_ref, h0_ref, o0_ref), (nd1_ref, h1_ref, o1_ref)]):
        parts = []
        for hp in range(HP):
            nd = nd_ref[hp, :, :]
            for hh in range(2):
                den = nd[:, 32 + hh:33 + hh] + 1e-16
                parts.append(nd[:, hh * 16:hh * 16 + 16] / den)
        agg = jnp.concatenate(parts, axis=1)
        gelu = agg * 0.5 * (1.0 + lax.erf(agg * (1.0 / math.sqrt(2.0))))
        o = jnp.dot(gelu, wa[t], precision=lax.Precision.HIGHEST) + ba[t:t + 1, :]
        s = sgm[0, t]
        o_ref[...] = s * o + (1.0 - s) * h_ref[...]


def _epilogue(h0, h1, nd0, nd1, wa, ba, sgm):
    return pl.pallas_call(
        _epilogue_body,
        grid=(GRID,),
        in_specs=[
            pl.BlockSpec((BN, HID), lambda i: (i, 0)),
            pl.BlockSpec((BN, HID), lambda i: (i, 0)),
            pl.BlockSpec((HP, BN, W_ROW), lambda i: (0, i, 0)),
            pl.BlockSpec((HP, BN, W_ROW), lambda i: (0, i, 0)),
            pl.BlockSpec((2, HID, HID), lambda i: (0, 0, 0)),
            pl.BlockSpec((2, HID), lambda i: (0, 0)),
            pl.BlockSpec((1, 2), lambda i: (0, 0)),
        ],
        out_specs=[
            pl.BlockSpec((BN, HID), lambda i: (i, 0)),
            pl.BlockSpec((BN, HID), lambda i: (i, 0)),
        ],
        out_shape=[jax.ShapeDtypeStruct((N, HID), _f32)] * 2,
    )(h0, h1, nd0, nd1, wa, ba, sgm)


def _head_body(h0_ref, w_ref, b_ref, out_ref):
    logits = jnp.dot(h0_ref[...], w_ref[...],
                     precision=lax.Precision.HIGHEST) + b_ref[...]
    m = jnp.max(logits, axis=-1, keepdims=True)
    z = logits - m
    lse = jnp.log(jnp.sum(jnp.exp(z), axis=-1, keepdims=True))
    out_ref[...] = z - lse


def _head(h0, lin_out_W, lin_out_b):
    return pl.pallas_call(
        _head_body,
        grid=(GRID,),
        in_specs=[
            pl.BlockSpec((BN, HID), lambda i: (i, 0)),
            pl.BlockSpec((HID, C), lambda i: (0, 0)),
            pl.BlockSpec((1, C), lambda i: (0, 0)),
        ],
        out_specs=pl.BlockSpec((BN, C), lambda i: (i, 0)),
        out_shape=jax.ShapeDtypeStruct((N, C), _f32),
    )(h0, lin_out_W, lin_out_b)


_EC_PER_TEC = E_PAD // NS
_NCHUNK = _EC_PER_TEC // CHUNK
_STRIPE = N_ACC // NS


def _edge_body(qd_ref, kt_ref, vt_ref, src_ref, dstg_ref, dsts_ref, z_ref,
               out_ref,
               acc, src_v, qidx_v, dsts_v, qrows, krows, vrows, rows_v,
               sq, sk, sv):
    c = lax.axis_index("c")
    tid = lax.axis_index("s")

    r0 = tid * _STRIPE
    pltpu.sync_copy(z_ref, acc.at[pl.ds(r0, _STRIPE)])
    plsc.subcore_barrier()

    base_e = tid * _EC_PER_TEC
    coff = c * N

    def chunk_body(j, _):
        eb = base_e + j * CHUNK
        pltpu.sync_copy(src_ref.at[pl.ds(eb, CHUNK)], src_v)
        pltpu.sync_copy(dstg_ref.at[pl.ds(eb, CHUNK)], qidx_v)
        pltpu.sync_copy(dsts_ref.at[pl.ds(eb, CHUNK)], dsts_v)
        for g in range(CHUNK // LANES):
            sl = pl.ds(g * LANES, LANES)
            qidx_v[sl] = qidx_v[sl] + coff
            src_v[sl] = src_v[sl] + coff
        dq = pltpu.async_copy(qd_ref.at[qidx_v], qrows, sq)
        dk = pltpu.async_copy(kt_ref.at[src_v], krows, sk)
        dv = pltpu.async_copy(vt_ref.at[src_v], vrows, sv)
        dq.wait()
        dk.wait()
        dv.wait()
        for g in range(CHUNK // LANES):
            rows = lax.iota(jnp.int32, LANES) + g * LANES
            acc0 = jnp.zeros((LANES,), _f32)
            acc1 = jnp.zeros((LANES,), _f32)
            for d in range(D):
                cd = jnp.full((LANES,), d, jnp.int32)
                cd1 = jnp.full((LANES,), D + d, jnp.int32)
                acc0 = acc0 + (plsc.load_gather(qrows, [rows, cd])
                               * plsc.load_gather(krows, [rows, cd]))
                acc1 = acc1 + (plsc.load_gather(qrows, [rows, cd1])
                               * plsc.load_gather(krows, [rows, cd1]))
            ex0 = jnp.exp(acc0)
            ex1 = jnp.exp(acc1)
            plsc.store_scatter(rows_v, [rows, jnp.full((LANES,), 32, jnp.int32)], ex0)
            plsc.store_scatter(rows_v, [rows, jnp.full((LANES,), 33, jnp.int32)], ex1)
            for col in range(2 * D):
                cc = jnp.full((LANES,), col, jnp.int32)
                vv = plsc.load_gather(vrows, [rows, cc])
                ex = ex0 if col < D else ex1
                plsc.store_scatter(rows_v, [rows, cc], vv * ex)
        pltpu.sync_copy(rows_v, acc.at[dsts_v], add=True)
        return 0

    lax.fori_loop(0, _NCHUNK, chunk_body, 0)
    plsc.subcore_barrier()
    pltpu.sync_copy(acc.at[pl.ds(r0, _STRIPE)], out_ref.at[c, pl.ds(r0, _STRIPE)])


@functools.partial(
    pl.kernel,
    out_type=jax.ShapeDtypeStruct((HP, N_ACC, W_ROW), _f32),
    mesh=plsc.VectorSubcoreMesh(core_axis_name="c", subcore_axis_name="s"),
    compiler_params=pltpu.CompilerParams(use_tc_tiling_on_sc=False,
                                         needs_layout_passes=False),
    scratch_types=[
        pltpu.VMEM_SHARED((N_ACC, W_ROW), _f32),
        pltpu.VMEM((CHUNK,), jnp.int32),
        pltpu.VMEM((CHUNK,), jnp.int32),
        pltpu.VMEM((CHUNK,), jnp.int32),
        pltpu.VMEM((CHUNK, 2 * D), _f32),
        pltpu.VMEM((CHUNK, 2 * D), _f32),
        pltpu.VMEM((CHUNK, 2 * D), _f32),
        pltpu.VMEM((CHUNK, W_ROW), _f32),
        pltpu.SemaphoreType.DMA,
        pltpu.SemaphoreType.DMA,
        pltpu.SemaphoreType.DMA,
    ],
)
def _edge_kernel(qd_ref, kt_ref, vt_ref, src_ref, dstg_ref, dsts_ref, z_ref,
                 out_ref, acc, src_v, qidx_v, dsts_v, qrows, krows, vrows,
                 rows_v, sq, sk, sv):
    _edge_body(qd_ref, kt_ref, vt_ref, src_ref, dstg_ref, dsts_ref, z_ref,
               out_ref, acc, src_v, qidx_v, dsts_v, qrows, krows, vrows,
               rows_v, sq, sk, sv)


def kernel(x_web, x_usr, edge_clicks, edge_rev, lin_in_W, lin_in_b, Wk, bk,
           Wq, bq, Wv, bv, Wa, ba, skip, a_rel, m_rel, p_rel, lin_out_W,
           lin_out_b):
    st = jnp.array([1, 0])
    Wk_f = jnp.einsum('ltxhd,lthde->ltxhe',
                      Wk[:, st].reshape(L, 2, HID, H, D), a_rel).reshape(L, 2, HID, HID)
    bk_f = jnp.einsum('lthd,lthde->lthe',
                      bk[:, st].reshape(L, 2, H, D), a_rel).reshape(L, 2, HID)
    Wv_f = jnp.einsum('ltxhd,lthde->ltxhe',
                      Wv[:, st].reshape(L, 2, HID, H, D), m_rel).reshape(L, 2, HID, HID)
    bv_f = jnp.einsum('lthd,lthde->lthe',
                      bv[:, st].reshape(L, 2, H, D), m_rel).reshape(L, 2, HID)
    scale = (p_rel / math.sqrt(D))[:, :, :, None]
    Wq_f = (Wq.reshape(L, 2, HID, H, D) * scale[:, :, None]).reshape(L, 2, HID, HID)
    bq_f = (bq.reshape(L, 2, H, D) * scale).reshape(L, 2, HID)

    w0cat = [jnp.concatenate([Wq_f[l, 0], Wk_f[l, 1], Wv_f[l, 1]], axis=1) for l in range(L)]
    b0cat = [jnp.concatenate([bq_f[l, 0], bk_f[l, 1], bv_f[l, 1]])[None, :] for l in range(L)]
    w1cat = [jnp.concatenate([Wq_f[l, 1], Wk_f[l, 0], Wv_f[l, 0]], axis=1) for l in range(L)]
    b1cat = [jnp.concatenate([bq_f[l, 1], bk_f[l, 0], bv_f[l, 0]])[None, :] for l in range(L)]
    sgm = jax.nn.sigmoid(skip)

    pad = E_PAD - E
    edges = []
    for e2 in (edge_clicks, edge_rev):
        src_p = jnp.pad(e2[0], (0, pad))
        dst_g = jnp.pad(e2[1], (0, pad))
        dst_s = jnp.pad(e2[1], (0, pad), constant_values=TRASH)
        edges.append((src_p, dst_g, dst_s))

    zeros_stripe = jnp.zeros((N_ACC // NS, W_ROW), _f32)
    h0, h1 = _inproj(x_web, x_usr, lin_in_W, lin_in_b)
    for l in range(L):
        qd0, kt1, vt1, qd1, kt0, vt0 = _qkv(
            h0, h1, w0cat[l], b0cat[l], w1cat[l], b1cat[l])
        nd0 = _edge_kernel(qd0.reshape(HP * N, 32), kt0.reshape(HP * N, 32),
                           vt0.reshape(HP * N, 32), *edges[0], zeros_stripe)
        nd1 = _edge_kernel(qd1.reshape(HP * N, 32), kt1.reshape(HP * N, 32),
                           vt1.reshape(HP * N, 32), *edges[1], zeros_stripe)
        h0, h1 = _epilogue(h0, h1, nd0[:, :N, :], nd1[:, :N, :],
                           Wa[l], ba[l], sgm[l:l + 1, :])
    return _head(h0, lin_out_W, lin_out_b[None, :])
